# baseline (speedup 1.0000x reference)
import jax
import jax.numpy as jnp
from jax import lax
from jax.experimental import pallas as pl
from jax.experimental.pallas import tpu as pltpu
from jax.experimental.pallas import tpu_sc as plsc

B = 16384
HL = 50
D = 32
DB = 16
NC, NS = 2, 16
NW = NC * NS
BPW = B // NW
CH = 128
NCH = BPW // CH
NF = 6
HCH = HL * NCH
PR = BPW // 4
PRB = BPW // 8

_mesh = plsc.VectorSubcoreMesh(core_axis_name="c", subcore_axis_name="s")


def _body(num_h, idx_h, histT_h, wb_h,
          Ec0, Ec1, Ec2, Ec3, Eb0, Eb1, Eh,
          o_num, o_c0, o_c1, o_c2, o_c3, o_b0, o_b1, o_h,
          idx_v, histT_v, num_v, wb_v, stgA, stgB,
          gA, gB, fbuf, bb0, bb1, acc, inv_v,
          sin, sgA, sgB, sout):
    cid = lax.axis_index("c")
    sid = lax.axis_index("s")
    wid = sid * NC + cid

    gbuf = (gA, gB)
    stg = (stgA, stgB)
    sg = (sgA, sgB)
    m3 = jnp.full((16,), 3, jnp.int32)
    m7 = jnp.full((16,), 7, jnp.int32)

    ins = [
        pltpu.async_copy(idx_h.at[pl.ds(wid * NF * NCH, NF * NCH)], idx_v, sin),
        pltpu.async_copy(histT_h.at[pl.ds(wid * HCH, HCH)], histT_v, sin),
        pltpu.async_copy(num_h.at[pl.ds(wid * PRB, PRB)], num_v, sin),
        pltpu.async_copy(wb_h, wb_v, sin),
    ]
    for cp in ins:
        cp.wait()

    def stage_rows(src_ref, row, shift, dst_ref):
        def sbody(v, carry):
            dst_ref[pl.ds(v * 16, 16)] = lax.shift_right_logical(
                src_ref[row, pl.ds(v * 16, 16)],
                jnp.full((16,), shift, jnp.int32))
            return carry
        lax.fori_loop(0, 8, sbody, 0)

    def fire(table, p):
        return pltpu.async_copy(table.at[stg[p]], gbuf[p], sg[p])

    stage_rows(idx_v, 0, 2, stgA)
    dA = fire(Ec0, 0)
    stage_rows(idx_v, 1, 2, stgB)
    dB = fire(Ec0, 1)

    wvec = [(wb_v[k // 4, pl.ds((k % 4) * 32, 16)],
             wb_v[k // 4, pl.ds((k % 4) * 32 + 16, 16)]) for k in range(13)]
    bv0 = wb_v[3, pl.ds(32, 16)]
    bv1 = wb_v[3, pl.ds(48, 16)]

    def num_body(rr, carry):
        for s in range(4):
            r = rr * 4 + s
            nv = num_v[lax.div(r, 8), pl.ds(lax.rem(r, 8) * 16, 16)]
            a0, a1 = bv0, bv1
            for k in range(13):
                sv = jnp.broadcast_to(nv[k], (16,))
                a0 = a0 + sv * wvec[k][0]
                a1 = a1 + sv * wvec[k][1]
            fbuf[rr, pl.ds(s * 32, 16)] = a0
            fbuf[rr, pl.ds(s * 32 + 16, 16)] = a1
        return carry

    lax.fori_loop(0, PR, num_body, 0)
    wprev = pltpu.async_copy(fbuf, o_num.at[pl.ds(wid * PR, PR)], sout)

    def extract_chunk(p, f, q, s3, outbuf):
        mask = m7 if s3 else m3
        w = DB if s3 else D
        def ebody(g, carry):
            iq = lax.bitwise_and(idx_v[f * NCH + q, pl.ds(g * 16, 16)], mask)
            for i in range(16):
                col = pl.multiple_of(iq[i] * w, 16)
                if s3:
                    r = q * (CH // 8) + g * 2 + i // 8
                    oc = (i % 8) * 16
                    outbuf[r, pl.ds(oc, 16)] = gbuf[p][g * 16 + i,
                                                       pl.ds(col, 16)]
                else:
                    r = q * (CH // 4) + g * 4 + i // 4
                    oc = (i % 4) * 32
                    outbuf[r, pl.ds(oc, 16)] = gbuf[p][g * 16 + i,
                                                       pl.ds(col, 16)]
                    outbuf[r, pl.ds(oc + 16, 16)] = gbuf[p][g * 16 + i,
                                                            pl.ds(col + 16, 16)]
            return carry
        lax.fori_loop(0, 8, ebody, 0)

    fields = [
        (Ec0, 2, False, fbuf, o_c0),
        (Ec1, 2, False, fbuf, o_c1),
        (Ec2, 2, False, fbuf, o_c2),
        (Ec3, 2, False, fbuf, o_c3),
        (Eb0, 3, True, bb0, o_b0),
        (Eb1, 3, True, bb1, o_b1),
    ]

    pend = [dA, dB]
    NT = NF * NCH
    for t in range(NT):
        f, q = t // NCH, t % NCH
        table, shift, s3, outbuf, outarr = fields[f]
        p = t % 2
        pend[p].wait()
        if q == 0:
            wprev.wait()
        extract_chunk(p, f, q, s3, outbuf)
        if t + 2 < NT:
            nf, nq = (t + 2) // NCH, (t + 2) % NCH
            ntable, nshift = fields[nf][0], fields[nf][1]
            stage_rows(idx_v, nf * NCH + nq, nshift, stg[p])
            pend[p] = fire(ntable, p)
        if q == NCH - 1:
            if s3:
                wprev = pltpu.async_copy(
                    outbuf, outarr.at[pl.ds(wid * PRB, PRB)], sout)
            else:
                wprev = pltpu.async_copy(
                    outbuf, outarr.at[pl.ds(wid * PR, PR)], sout)

    stage_rows(histT_v, 0, 2, stgA)
    fire(Eh, 0)
    stage_rows(histT_v, 1, 2, stgB)
    fire(Eh, 1)

    def zero_body(r, carry):
        z = jnp.zeros((16,), jnp.float32)
        for v in range(8):
            acc[r, pl.ds(v * 16, 16)] = z
        return carry

    lax.fori_loop(0, PR, zero_body, 0)

    def hist_step(k, carry):
        for p in range(2):
            c = 2 * k + p
            pltpu.make_async_copy(Eh.at[pl.ds(0, CH)], gbuf[p], sg[p]).wait()
            roff4 = pl.multiple_of(lax.rem(c, NCH) * (CH // 4), CH // 4)

            def abody(g, carry2):
                iq = lax.bitwise_and(histT_v[c, pl.ds(g * 16, 16)], m3)
                for i in range(16):
                    col = pl.multiple_of(iq[i] * D, 16)
                    rr = roff4 + g * 4 + i // 4
                    oc = (i % 4) * 32
                    a0 = gbuf[p][g * 16 + i, pl.ds(col, 16)]
                    a1 = gbuf[p][g * 16 + i, pl.ds(col + 16, 16)]
                    plsc.addupdate(acc.at[rr, pl.ds(oc, 16)], a0)
                    plsc.addupdate(acc.at[rr, pl.ds(oc + 16, 16)], a1)
                return carry2

            lax.fori_loop(0, 8, abody, 0)

            @pl.when(c + 2 < HCH)
            def _():
                stage_rows(histT_v, c + 2, 2, stg[p])
                fire(Eh, p)
        return carry

    lax.fori_loop(0, HCH // 2, hist_step, 0)

    def cnt_body(g, carry):
        q = lax.div(g, CH // 16)
        off = lax.rem(g, CH // 16) * 16
        c = jnp.zeros((16,), jnp.float32)
        zi = jnp.zeros((16,), jnp.int32)
        one = jnp.full((16,), 1.0, jnp.float32)
        zf = jnp.zeros((16,), jnp.float32)
        for j in range(HL):
            iv = histT_v[j * NCH + q, pl.ds(off, 16)]
            c = c + jnp.where(iv != zi, one, zf)
        inv_v[pl.ds(g * 16, 16)] = one / jnp.maximum(
            c, jnp.full((16,), 1e-6, jnp.float32))
        return carry

    lax.fori_loop(0, BPW // 16, cnt_body, 0)

    def scale_body(g, carry):
        iv = inv_v[pl.ds(g * 16, 16)]
        for i in range(16):
            sv = jnp.broadcast_to(iv[i], (16,))
            rr = g * 4 + i // 4
            oc = (i % 4) * 32
            acc[rr, pl.ds(oc, 16)] = acc[rr, pl.ds(oc, 16)] * sv
            acc[rr, pl.ds(oc + 16, 16)] = acc[rr, pl.ds(oc + 16, 16)] * sv
        return carry

    lax.fori_loop(0, BPW // 16, scale_body, 0)
    wh = pltpu.async_copy(acc, o_h.at[pl.ds(wid * PR, PR)], sout)
    wprev.wait()
    wh.wait()


_WIDE = jax.ShapeDtypeStruct((B // 4, 128), jnp.float32)
_NARROW = jax.ShapeDtypeStruct((B // 8, 128), jnp.float32)

_encode = pl.kernel(
    _body,
    out_type=[_WIDE, _WIDE, _WIDE, _WIDE, _WIDE, _NARROW, _NARROW, _WIDE],
    mesh=_mesh,
    scratch_types=[
        pltpu.VMEM((NF * NCH, CH), jnp.int32),
        pltpu.VMEM((HCH, CH), jnp.int32),
        pltpu.VMEM((PRB, 128), jnp.float32),
        pltpu.VMEM((4, 128), jnp.float32),
        pltpu.VMEM((CH,), jnp.int32),
        pltpu.VMEM((CH,), jnp.int32),
        pltpu.VMEM((CH, CH), jnp.float32),
        pltpu.VMEM((CH, CH), jnp.float32),
        pltpu.VMEM((PR, 128), jnp.float32),
        pltpu.VMEM((PRB, 128), jnp.float32),
        pltpu.VMEM((PRB, 128), jnp.float32),
        pltpu.VMEM((PR, 128), jnp.float32),
        pltpu.VMEM((BPW,), jnp.float32),
        pltpu.SemaphoreType.DMA,
        pltpu.SemaphoreType.DMA,
        pltpu.SemaphoreType.DMA,
        pltpu.SemaphoreType.DMA,
    ],
)


def kernel(numeric, cat_0, cat_1, cat_2, cat_3, bkt_0, bkt_1, hist_items,
           W_num, b_num, E_cat_0, E_cat_1, E_cat_2, E_cat_3,
           E_bkt_0, E_bkt_1, E_hist):
    num_p = jnp.pad(numeric, ((0, 0), (0, 3))).reshape(B * 16 // 128, 128)
    idx_all = jnp.stack([cat_0, cat_1, cat_2, cat_3, bkt_0, bkt_1])
    idx_all = idx_all.reshape(NF, NW, BPW).transpose(1, 0, 2)
    idx_all = idx_all.reshape(NW * NF * NCH, CH)
    hist_t = jnp.transpose(hist_items).reshape(HL, NW, BPW).transpose(1, 0, 2)
    hist_t = hist_t.reshape(NW * HCH, CH)
    wb = jnp.concatenate([jnp.transpose(W_num), b_num[None, :],
                          jnp.zeros((2, 32), jnp.float32)], axis=0)
    wb = wb.reshape(4, 128)
    outs = _encode(num_p, idx_all, hist_t, wb,
                   E_cat_0.reshape(-1, 128), E_cat_1.reshape(-1, 128),
                   E_cat_2.reshape(-1, 128), E_cat_3.reshape(-1, 128),
                   E_bkt_0.reshape(-1, 128), E_bkt_1.reshape(-1, 128),
                   E_hist.reshape(-1, 128))
    widths = (D, D, D, D, D, DB, DB, D)
    flat = [o.reshape(B, w) for o, w in zip(outs, widths)]
    return jnp.concatenate(flat, axis=-1)
